# baseline (speedup 1.0000x reference)
import jax
import jax.numpy as jnp
from jax.experimental import pallas as pl
from jax.experimental.pallas import tpu as pltpu

N = 1024
FD = 64
K2 = 8
RBLK = N // K2


def _mm(a, b):
    return jax.lax.dot_general(a, b, (((1,), (0,)), ((), ())),
                               preferred_element_type=jnp.float32)


def _mmT(a, b):
    return jax.lax.dot_general(a, b, (((1,), (1,)), ((), ())),
                               preferred_element_type=jnp.float32)


def _mTm(a, b):
    return jax.lax.dot_general(a, b, (((0,), (0,)), ((), ())),
                               preferred_element_type=jnp.float32)


def _gcn_branch(m, rbias, w1, b1, w2, b2):
    mask = ((m + rbias) != 0.0).astype(jnp.float32)
    ones_col = jnp.ones((N, 1), jnp.float32)
    deg = _mTm(mask, ones_col) + 1.0
    dinv = jax.lax.rsqrt(deg)

    x1 = dinv * w1
    t1 = _mTm(mask, x1)
    h1 = jnp.maximum(dinv * (t1 + x1) + b1, 0.0)

    y = dinv * _mm(h1, w2)
    t2 = _mTm(mask, y)
    return dinv * (t2 + y) + b2


def _gcn_core(md_ref, w1d_ref, b1d_ref, w2d_ref, b2d_ref,
              ma_ref, rba_ref, w1a_ref, b1a_ref, w2a_ref, b2a_ref,
              h2d_ref, h2a_ref):
    zrow = jnp.zeros((1, N), jnp.float32)
    h2d_ref[...] = _gcn_branch(md_ref[...], zrow, w1d_ref[...], b1d_ref[...],
                               w2d_ref[...], b2d_ref[...])
    h2a_ref[...] = _gcn_branch(ma_ref[...], rba_ref[...], w1a_ref[...],
                               b1a_ref[...], w2a_ref[...], b2a_ref[...])


def _readout(h2d_ref, h2a_ref, lwd_ref, lwa_ref, od_ref, oa_ref, pd, pa):
    k = pl.program_id(0)

    @pl.when(k == 0)
    def _init():
        pd[...] = jnp.zeros_like(pd)
        pa[...] = jnp.zeros_like(pa)

    pd[...] += _mTm(h2d_ref[...], lwd_ref[...])
    pa[...] += _mTm(h2a_ref[...], lwa_ref[...])

    @pl.when(k == K2 - 1)
    def _fin():
        ci = jax.lax.broadcasted_iota(jnp.int32, (FD, FD * FD), 1)
        ri = jax.lax.broadcasted_iota(jnp.int32, (FD, FD * FD), 0)
        colmask = ((ci // FD) == ri).astype(jnp.float32)
        rr = jax.lax.broadcasted_iota(jnp.int32, (FD * FD, FD), 0)
        rc = jax.lax.broadcasted_iota(jnp.int32, (FD * FD, FD), 1)
        fold = ((rr % FD) == rc).astype(jnp.float32)
        ones_col = jnp.ones((FD, 1), jnp.float32)
        od_ref[...] = _mm(_mTm(ones_col, pd[...] * colmask), fold)
        oa_ref[...] = _mm(_mTm(ones_col, pa[...] * colmask), fold)


def _tail(od_ref, oa_ref, dlb_ref, alb_ref,
          gfvw_ref, gfvb_ref, gfow_ref, gfob_ref,
          savw_ref, savb_ref, saow_ref, saob_ref,
          hol_ref, hw1_ref, hb1_ref, hw2_ref, hb2_ref,
          tfvw_ref, tfvb_ref, tfow_ref, tfob_ref,
          cavw_ref, cavb_ref, ffw_ref, ffb_ref, out_ref):
    od = od_ref[...] + dlb_ref[...]
    oa = oa_ref[...] + alb_ref[...]
    fusion = _mmT(_mmT(od, gfvw_ref[...]) + gfvb_ref[...],
                  gfow_ref[...]) + gfob_ref[...]
    cat = jnp.concatenate([fusion, oa], axis=1)
    g = _mm(_mm(cat, savw_ref[...]) + savb_ref[...],
            saow_ref[...]) + saob_ref[...]
    hh = jnp.maximum(_mm(hol_ref[...], hw1_ref[...]) + hb1_ref[...], 0.0)
    feat = _mm(hh, hw2_ref[...]) + hb2_ref[...]
    f = _mmT(_mmT(feat, tfvw_ref[...]) + tfvb_ref[...],
             tfow_ref[...]) + tfob_ref[...]
    t = _mm(f, cavw_ref[...]) + cavb_ref[...]
    out_ref[...] = _mm(jnp.concatenate([g, t], axis=1),
                       ffw_ref[...]) + ffb_ref[...]


def _row(v):
    return v.reshape(1, -1)


def kernel(adj_matrix, con_matrix, dis_matrix, sim_matrix, tou, time, day,
           holiday, weather, params):
    p = params

    h2d, h2a = pl.pallas_call(
        _gcn_core,
        out_shape=[jax.ShapeDtypeStruct((N, FD), jnp.float32),
                   jax.ShapeDtypeStruct((N, FD), jnp.float32)],
    )(dis_matrix, p['gcn_dis_w1'], _row(p['gcn_dis_b1']),
      p['gcn_dis_w2'], _row(p['gcn_dis_b2']),
      p['ada_l1w'], _row(p['ada_l1b']), p['ada_w1'],
      _row(p['ada_b1']), p['ada_w2'], _row(p['ada_b2']))

    od, oa = pl.pallas_call(
        _readout,
        grid=(K2,),
        in_specs=[
            pl.BlockSpec((RBLK, FD), lambda k: (k, 0)),
            pl.BlockSpec((RBLK, FD), lambda k: (k, 0)),
            pl.BlockSpec((RBLK, FD * FD), lambda k: (k, 0)),
            pl.BlockSpec((RBLK, FD * FD), lambda k: (k, 0)),
        ],
        out_specs=[pl.BlockSpec((1, FD), lambda k: (0, 0)),
                   pl.BlockSpec((1, FD), lambda k: (0, 0))],
        out_shape=[jax.ShapeDtypeStruct((1, FD), jnp.float32),
                   jax.ShapeDtypeStruct((1, FD), jnp.float32)],
        scratch_shapes=[pltpu.VMEM((FD, FD * FD), jnp.float32),
                        pltpu.VMEM((FD, FD * FD), jnp.float32)],
    )(h2d, h2a,
      p['gcn_dis_lw'].reshape(N, FD * FD), p['ada_lw'].reshape(N, FD * FD))

    out = pl.pallas_call(
        _tail,
        out_shape=jax.ShapeDtypeStruct((1, FD), jnp.float32),
    )(od, oa,
      _row(p['gcn_dis_lb']), _row(p['ada_lb']),
      p['gf_inw'][2 * FD:], _row(p['gf_inb'][2 * FD:]),
      p['gf_outw'], _row(p['gf_outb']),
      p['sa_vw'], _row(p['sa_vb']), p['sa_ow'], _row(p['sa_ob']),
      _row(holiday), p['mlp_holiday_w1'], _row(p['mlp_holiday_b1']),
      p['mlp_holiday_w2'], _row(p['mlp_holiday_b2']),
      p['tf_inw'][2 * FD:], _row(p['tf_inb'][2 * FD:]),
      p['tf_outw'], _row(p['tf_outb']),
      p['ca_vw'], _row(p['ca_vb']),
      p['ff_w'], _row(p['ff_b']))
    return out


# speedup vs baseline: 1.0081x; 1.0081x over previous
import jax
import jax.numpy as jnp
from jax.experimental import pallas as pl
from jax.experimental.pallas import tpu as pltpu

N = 1024
FD = 64
K2 = 8
RBLK = N // K2


def _mm(a, b):
    return jax.lax.dot_general(a, b, (((1,), (0,)), ((), ())),
                               preferred_element_type=jnp.float32)


def _mmT(a, b):
    return jax.lax.dot_general(a, b, (((1,), (1,)), ((), ())),
                               preferred_element_type=jnp.float32)


def _mTm(a, b):
    return jax.lax.dot_general(a, b, (((0,), (0,)), ((), ())),
                               preferred_element_type=jnp.float32)


def _gcn_branch(m, rbias, w1, b1, w2, b2):
    mask = ((m + rbias) != 0.0).astype(jnp.float32)
    ones_col = jnp.ones((N, 1), jnp.float32)
    deg = _mTm(mask, ones_col) + 1.0
    dinv = jax.lax.rsqrt(deg)

    x1 = dinv * w1
    t1 = _mTm(mask, x1)
    h1 = jnp.maximum(dinv * (t1 + x1) + b1, 0.0)

    y = dinv * _mm(h1, w2)
    t2 = _mTm(mask, y)
    return dinv * (t2 + y) + b2


def _gcn_core(md_ref, w1d_ref, b1d_ref, w2d_ref, b2d_ref,
              ma_ref, rba_ref, w1a_ref, b1a_ref, w2a_ref, b2a_ref,
              h2d_ref, h2a_ref):
    zrow = jnp.zeros((1, N), jnp.float32)
    h2d_ref[...] = _gcn_branch(md_ref[...], zrow, w1d_ref[...], b1d_ref[...],
                               w2d_ref[...], b2d_ref[...])
    h2a_ref[...] = _gcn_branch(ma_ref[...], rba_ref[...], w1a_ref[...],
                               b1a_ref[...], w2a_ref[...], b2a_ref[...])


NSTRIP = 4
SW = (FD * FD) // NSTRIP


def _readout_tail(h2d_ref, h2a_ref, lwd_hbm, lwa_hbm,
                  dlb_ref, alb_ref,
                  gfvw_ref, gfvb_ref, gfow_ref, gfob_ref,
                  savw_ref, savb_ref, saow_ref, saob_ref,
                  hol_ref, hw1_ref, hb1_ref, hw2_ref, hb2_ref,
                  tfvw_ref, tfvb_ref, tfow_ref, tfob_ref,
                  cavw_ref, cavb_ref, ffw_ref, ffb_ref,
                  out_ref, bufd, bufa, pd, pa, sems):
    def copies(k, par):
        out = []
        for b, (hbm, buf) in enumerate(((lwd_hbm, bufd), (lwa_hbm, bufa))):
            for j in range(NSTRIP):
                out.append(pltpu.make_async_copy(
                    hbm.at[pl.ds(k * RBLK, RBLK), pl.ds(j * SW, SW)],
                    buf.at[par, :, pl.ds(j * SW, SW)],
                    sems.at[par, b, j]))
        return out

    for c in copies(0, 0):
        c.start()
    pd[...] = jnp.zeros_like(pd)
    pa[...] = jnp.zeros_like(pa)
    for k in range(K2):
        par = k % 2
        if k + 1 < K2:
            for c in copies(k + 1, (k + 1) % 2):
                c.start()
        for c in copies(k, par):
            c.wait()
        h2d_blk = h2d_ref[pl.ds(k * RBLK, RBLK), :]
        h2a_blk = h2a_ref[pl.ds(k * RBLK, RBLK), :]
        pd[...] += _mTm(h2d_blk, bufd[par])
        pa[...] += _mTm(h2a_blk, bufa[par])

    ci = jax.lax.broadcasted_iota(jnp.int32, (FD, FD * FD), 1)
    ri = jax.lax.broadcasted_iota(jnp.int32, (FD, FD * FD), 0)
    colmask = ((ci // FD) == ri).astype(jnp.float32)
    rr = jax.lax.broadcasted_iota(jnp.int32, (FD * FD, FD), 0)
    rc = jax.lax.broadcasted_iota(jnp.int32, (FD * FD, FD), 1)
    fold = ((rr % FD) == rc).astype(jnp.float32)
    ones_col = jnp.ones((FD, 1), jnp.float32)
    od = _mm(_mTm(ones_col, pd[...] * colmask), fold) + dlb_ref[...]
    oa = _mm(_mTm(ones_col, pa[...] * colmask), fold) + alb_ref[...]

    fusion = _mmT(_mmT(od, gfvw_ref[...]) + gfvb_ref[...],
                  gfow_ref[...]) + gfob_ref[...]
    cat = jnp.concatenate([fusion, oa], axis=1)
    g = _mm(_mm(cat, savw_ref[...]) + savb_ref[...],
            saow_ref[...]) + saob_ref[...]
    hh = jnp.maximum(_mm(hol_ref[...], hw1_ref[...]) + hb1_ref[...], 0.0)
    feat = _mm(hh, hw2_ref[...]) + hb2_ref[...]
    f = _mmT(_mmT(feat, tfvw_ref[...]) + tfvb_ref[...],
             tfow_ref[...]) + tfob_ref[...]
    t = _mm(f, cavw_ref[...]) + cavb_ref[...]
    out_ref[...] = _mm(jnp.concatenate([g, t], axis=1),
                       ffw_ref[...]) + ffb_ref[...]


def _row(v):
    return v.reshape(1, -1)


def kernel(adj_matrix, con_matrix, dis_matrix, sim_matrix, tou, time, day,
           holiday, weather, params):
    p = params

    h2d, h2a = pl.pallas_call(
        _gcn_core,
        out_shape=[jax.ShapeDtypeStruct((N, FD), jnp.float32),
                   jax.ShapeDtypeStruct((N, FD), jnp.float32)],
    )(dis_matrix, p['gcn_dis_w1'], _row(p['gcn_dis_b1']),
      p['gcn_dis_w2'], _row(p['gcn_dis_b2']),
      p['ada_l1w'], _row(p['ada_l1b']), p['ada_w1'],
      _row(p['ada_b1']), p['ada_w2'], _row(p['ada_b2']))

    vm = pl.BlockSpec(memory_space=pltpu.MemorySpace.VMEM)
    hbm = pl.BlockSpec(memory_space=pltpu.MemorySpace.HBM)
    out = pl.pallas_call(
        _readout_tail,
        in_specs=[vm, vm, hbm, hbm] + [vm] * 23,
        out_shape=jax.ShapeDtypeStruct((1, FD), jnp.float32),
        scratch_shapes=[pltpu.VMEM((2, RBLK, FD * FD), jnp.float32),
                        pltpu.VMEM((2, RBLK, FD * FD), jnp.float32),
                        pltpu.VMEM((FD, FD * FD), jnp.float32),
                        pltpu.VMEM((FD, FD * FD), jnp.float32),
                        pltpu.SemaphoreType.DMA((2, 2, NSTRIP))],
    )(h2d, h2a,
      p['gcn_dis_lw'].reshape(N, FD * FD), p['ada_lw'].reshape(N, FD * FD),
      _row(p['gcn_dis_lb']), _row(p['ada_lb']),
      p['gf_inw'][2 * FD:], _row(p['gf_inb'][2 * FD:]),
      p['gf_outw'], _row(p['gf_outb']),
      p['sa_vw'], _row(p['sa_vb']), p['sa_ow'], _row(p['sa_ob']),
      _row(holiday), p['mlp_holiday_w1'], _row(p['mlp_holiday_b1']),
      p['mlp_holiday_w2'], _row(p['mlp_holiday_b2']),
      p['tf_inw'][2 * FD:], _row(p['tf_inb'][2 * FD:]),
      p['tf_outw'], _row(p['tf_outb']),
      p['ca_vw'], _row(p['ca_vb']),
      p['ff_w'], _row(p['ff_b']))
    return out


# speedup vs baseline: 1.2096x; 1.1999x over previous
import jax
import jax.numpy as jnp
from jax.experimental import pallas as pl
from jax.experimental.pallas import tpu as pltpu

N = 1024
FD = 64
K_CHUNKS = 8
CHUNK = (N * FD) // K_CHUNKS


def _mm(a, b):
    return jax.lax.dot_general(a, b, (((1,), (0,)), ((), ())),
                               preferred_element_type=jnp.float32)


def _mmT(a, b):
    return jax.lax.dot_general(a, b, (((1,), (1,)), ((), ())),
                               preferred_element_type=jnp.float32)


def _mTm(a, b):
    return jax.lax.dot_general(a, b, (((0,), (0,)), ((), ())),
                               preferred_element_type=jnp.float32)


def _gcn_branch(m, rbias, w1, b1, w2, b2):
    mask = ((m + rbias) != 0.0).astype(jnp.float32)
    ones_col = jnp.ones((N, 1), jnp.float32)
    deg = _mTm(mask, ones_col) + 1.0
    dinv = jax.lax.rsqrt(deg)
    x1 = dinv * w1
    t1 = _mTm(mask, x1)
    h1 = jnp.maximum(dinv * (t1 + x1) + b1, 0.0)
    y = dinv * _mm(h1, w2)
    t2 = _mTm(mask, y)
    return dinv * (t2 + y) + b2


def _gcn_core(md_ref, w1d_ref, b1d_ref, w2d_ref, b2d_ref,
              ma_ref, rba_ref, w1a_ref, b1a_ref, w2a_ref, b2a_ref,
              h2d_ref, h2a_ref):
    zrow = jnp.zeros((1, N), jnp.float32)
    h2d_ref[...] = _gcn_branch(md_ref[...], zrow, w1d_ref[...], b1d_ref[...],
                               w2d_ref[...], b2d_ref[...])
    h2a_ref[...] = _gcn_branch(ma_ref[...], rba_ref[...], w1a_ref[...],
                               b1a_ref[...], w2a_ref[...], b2a_ref[...])


def _readout_tail(h2d_ref, h2a_ref, lwd_ref, lwa_ref,
                  dlb_ref, alb_ref,
                  gfvw_ref, gfvb_ref, gfow_ref, gfob_ref,
                  savw_ref, savb_ref, saow_ref, saob_ref,
                  hol_ref, hw1_ref, hb1_ref, hw2_ref, hb2_ref,
                  tfvw_ref, tfvb_ref, tfow_ref, tfob_ref,
                  cavw_ref, cavb_ref, ffw_ref, ffb_ref,
                  out_ref, accd, acca):
    k = pl.program_id(0)

    @pl.when(k == 0)
    def _init():
        accd[...] = jnp.zeros_like(accd)
        acca[...] = jnp.zeros_like(acca)

    accd[...] += _mm(h2d_ref[...], lwd_ref[...])
    acca[...] += _mm(h2a_ref[...], lwa_ref[...])

    @pl.when(k == K_CHUNKS - 1)
    def _tail():
        od = accd[...] + dlb_ref[...]
        oa = acca[...] + alb_ref[...]
        fusion = _mmT(_mmT(od, gfvw_ref[...]) + gfvb_ref[...],
                      gfow_ref[...]) + gfob_ref[...]
        cat = jnp.concatenate([fusion, oa], axis=1)
        g = _mm(_mm(cat, savw_ref[...]) + savb_ref[...],
                saow_ref[...]) + saob_ref[...]
        hh = jnp.maximum(_mm(hol_ref[...], hw1_ref[...]) + hb1_ref[...], 0.0)
        feat = _mm(hh, hw2_ref[...]) + hb2_ref[...]
        f = _mmT(_mmT(feat, tfvw_ref[...]) + tfvb_ref[...],
                 tfow_ref[...]) + tfob_ref[...]
        t = _mm(f, cavw_ref[...]) + cavb_ref[...]
        out_ref[...] = _mm(jnp.concatenate([g, t], axis=1),
                           ffw_ref[...]) + ffb_ref[...]


def _row(v):
    return v.reshape(1, -1)


def kernel(adj_matrix, con_matrix, dis_matrix, sim_matrix, tou, time, day,
           holiday, weather, params):
    p = params

    h2d, h2a = pl.pallas_call(
        _gcn_core,
        out_shape=[jax.ShapeDtypeStruct((N, FD), jnp.float32),
                   jax.ShapeDtypeStruct((N, FD), jnp.float32)],
    )(dis_matrix, p['gcn_dis_w1'], _row(p['gcn_dis_b1']),
      p['gcn_dis_w2'], _row(p['gcn_dis_b2']),
      p['ada_l1w'], _row(p['ada_l1b']), p['ada_w1'],
      _row(p['ada_b1']), p['ada_w2'], _row(p['ada_b2']))

    full = lambda shape: pl.BlockSpec(shape, lambda k: (0, 0))
    out = pl.pallas_call(
        _readout_tail,
        grid=(K_CHUNKS,),
        in_specs=[
            pl.BlockSpec((1, CHUNK), lambda k: (0, k)),
            pl.BlockSpec((1, CHUNK), lambda k: (0, k)),
            pl.BlockSpec((CHUNK, FD), lambda k: (k, 0)),
            pl.BlockSpec((CHUNK, FD), lambda k: (k, 0)),
            full((1, FD)), full((1, FD)),
            full((FD, FD)), full((1, FD)), full((FD, FD)), full((1, FD)),
            full((2 * FD, 2 * FD)), full((1, 2 * FD)),
            full((2 * FD, FD)), full((1, FD)),
            full((1, 512)), full((512, FD)), full((1, FD)),
            full((FD, FD)), full((1, FD)),
            full((FD, FD)), full((1, FD)), full((FD, FD)), full((1, FD)),
            full((FD, FD)), full((1, FD)),
            full((2 * FD, FD)), full((1, FD)),
        ],
        out_specs=pl.BlockSpec((1, FD), lambda k: (0, 0)),
        out_shape=jax.ShapeDtypeStruct((1, FD), jnp.float32),
        scratch_shapes=[pltpu.VMEM((1, FD), jnp.float32),
                        pltpu.VMEM((1, FD), jnp.float32)],
    )(
        h2d.reshape(1, N * FD), h2a.reshape(1, N * FD),
        p['gcn_dis_lw'], p['ada_lw'],
        _row(p['gcn_dis_lb']), _row(p['ada_lb']),
        p['gf_inw'][2 * FD:], _row(p['gf_inb'][2 * FD:]),
        p['gf_outw'], _row(p['gf_outb']),
        p['sa_vw'], _row(p['sa_vb']), p['sa_ow'], _row(p['sa_ob']),
        _row(holiday), p['mlp_holiday_w1'], _row(p['mlp_holiday_b1']),
        p['mlp_holiday_w2'], _row(p['mlp_holiday_b2']),
        p['tf_inw'][2 * FD:], _row(p['tf_inb'][2 * FD:]),
        p['tf_outw'], _row(p['tf_outb']),
        p['ca_vw'], _row(p['ca_vb']),
        p['ff_w'], _row(p['ff_b']),
    )
    return out
